# baseline (speedup 1.0000x reference)
import functools

import jax
import jax.numpy as jnp
from jax import lax
from jax.experimental import pallas as pl
from jax.experimental.pallas import tpu as pltpu
from jax.experimental.pallas import tpu_sc as plsc

BATCH = 4096
HIST = 20
DIM = 64
TOTAL = BATCH * HIST
NUM_EMB = 1000000
NUM_CORES = 2
NUM_SUBCORES = 16
NW = NUM_CORES * NUM_SUBCORES
PER_W = TOTAL // NW
CHUNK = 128
LANES = 16

TCH = 256
STEPS = (NUM_EMB // TCH) // NW
MAIN = STEPS * NW * TCH
TAIL = NUM_EMB - MAIN
N_CHUNK = PER_W // CHUNK
PACKED = NUM_EMB // 2
N_PAIR = N_CHUNK // 2

_mesh = plsc.VectorSubcoreMesh(core_axis_name="c", subcore_axis_name="s")
_params = pltpu.CompilerParams(use_tc_tiling_on_sc=True,
                               needs_layout_passes=False)


@functools.partial(
    pl.kernel,
    mesh=_mesh,
    out_type=jax.ShapeDtypeStruct((PACKED, 128), jnp.float32),
    scratch_types=[
        pltpu.VMEM((DIM, TCH), jnp.float32),
        pltpu.VMEM((DIM, TCH), jnp.float32),
        pltpu.VMEM((TCH // 2, 128), jnp.float32),
        pltpu.VMEM((TCH // 2, 128), jnp.float32),
        pltpu.SemaphoreType.DMA,
        pltpu.SemaphoreType.DMA,
        pltpu.SemaphoreType.DMA,
        pltpu.SemaphoreType.DMA,
    ],
    compiler_params=_params,
)
def _transpose_kernel(tT_hbm, tail_hbm, scratch_hbm,
                      tin0, tin1, tout0, tout1, si0, si1, so0, so1):
    wid = lax.axis_index("s") * NUM_CORES + lax.axis_index("c")
    lo = wid * STEPS
    iota = lax.iota(jnp.int32, LANES)
    one = jnp.full((LANES,), 1, jnp.int32)
    dimv = jnp.full((LANES,), DIM, jnp.int32)

    def fire_in(step, tin, si):
        pltpu.async_copy(tT_hbm.at[:, pl.ds(step * TCH, TCH)], tin, si)

    def wait_in(tin, si):
        pltpu.make_async_copy(tT_hbm.at[:, pl.ds(0, TCH)], tin, si).wait()

    def fire_out(step, tout, so):
        pltpu.async_copy(
            tout, scratch_hbm.at[pl.ds(step * (TCH // 2), TCH // 2)], so)

    def wait_out(tout, so):
        pltpu.make_async_copy(tout, scratch_hbm.at[pl.ds(0, TCH // 2)],
                              so).wait()

    def shuffle(tin, tout):
        @plsc.parallel_loop(0, TCH, step=LANES, unroll=4)
        def _(i0):
            for f0 in range(0, DIM, LANES):
                rowv = f0 + iota
                for k in range(LANES):
                    colv = i0 + lax.bitwise_and(iota + k, LANES - 1)
                    vals = plsc.load_gather(tin, [rowv, colv])
                    prow = lax.shift_right_logical(colv, one)
                    pcol = lax.mul(lax.bitwise_and(colv, one), dimv) + rowv
                    plsc.store_scatter(tout, [prow, pcol], vals)

    fire_in(lo, tin0, si0)
    fire_in(lo + 1, tin1, si1)

    def body(g, _):
        s0 = lo + 2 * g
        wait_in(tin0, si0)

        @pl.when(g > 0)
        def _():
            wait_out(tout0, so0)

        shuffle(tin0, tout0)
        fire_out(s0, tout0, so0)
        fire_in(s0 + 2, tin0, si0)

        wait_in(tin1, si1)

        @pl.when(g > 0)
        def _():
            wait_out(tout1, so1)

        shuffle(tin1, tout1)
        fire_out(s0 + 1, tout1, so1)
        fire_in(s0 + 3, tin1, si1)
        return 0

    lax.fori_loop(0, STEPS // 2, body, 0)
    wait_in(tin0, si0)
    wait_in(tin1, si1)
    wait_out(tout0, so0)
    wait_out(tout1, so1)

    @pl.when(wid == NW - 1)
    def _():
        half = TCH // 2
        for t in range(2):
            pltpu.sync_copy(tail_hbm.at[pl.ds(t * half, half)], tout0)
            pltpu.sync_copy(
                tout0, scratch_hbm.at[pl.ds(MAIN // 2 + t * half, half)])
        rest = TAIL // 2 - 2 * half
        pltpu.sync_copy(tail_hbm.at[pl.ds(2 * half, rest)],
                        tout0.at[pl.ds(0, rest)])
        pltpu.sync_copy(tout0.at[pl.ds(0, rest)],
                        scratch_hbm.at[pl.ds(MAIN // 2 + 2 * half, rest)])


@functools.partial(
    pl.kernel,
    mesh=_mesh,
    out_type=jax.ShapeDtypeStruct((TOTAL, 128), jnp.float32),
    scratch_types=[
        pltpu.VMEM((PER_W,), jnp.int32),
        pltpu.VMEM((PER_W,), jnp.int32),
        pltpu.VMEM((CHUNK, 128), jnp.float32),
        pltpu.VMEM((CHUNK, 128), jnp.float32),
        pltpu.VMEM((CHUNK, 128), jnp.float32),
        pltpu.VMEM((CHUNK, 128), jnp.float32),
        pltpu.SemaphoreType.DMA,
        pltpu.SemaphoreType.DMA,
        pltpu.SemaphoreType.DMA,
        pltpu.SemaphoreType.DMA,
    ],
    compiler_params=_params,
)
def _gather_kernel(idx_hbm, lines_hbm, out_hbm, idx_v, pidx_v, b0, b1,
                   ob0, ob1, sg0, sg1, so0, so1):
    wid = lax.axis_index("s") * NUM_CORES + lax.axis_index("c")
    base = wid * PER_W
    iota = lax.iota(jnp.int32, LANES)
    one = jnp.full((LANES,), 1, jnp.int32)
    dimv = jnp.full((LANES,), DIM, jnp.int32)
    pltpu.sync_copy(idx_hbm.at[wid], idx_v)

    @plsc.parallel_loop(0, PER_W, step=LANES, unroll=4)
    def _(i):
        pidx_v[pl.ds(i, LANES)] = lax.shift_right_logical(
            idx_v[pl.ds(i, LANES)], one)

    def select(j, buf, ob):
        off = j * CHUNK

        @plsc.parallel_loop(0, CHUNK, step=LANES, unroll=4)
        def _(r0):
            rows = r0 + iota
            hvec = lax.mul(
                lax.bitwise_and(idx_v[pl.ds(off + r0, LANES)], one), dimv)
            for f0 in range(0, DIM, LANES):
                for k in range(LANES):
                    fv = f0 + lax.bitwise_and(iota + k, LANES - 1)
                    vals = plsc.load_gather(buf, [rows, hvec + fv])
                    plsc.store_scatter(ob, [rows, fv], vals)

    def fire_g(j, buf, sg):
        pltpu.async_copy(lines_hbm.at[pidx_v.at[pl.ds(j * CHUNK, CHUNK)]],
                         buf, sg)

    def wait_g(buf, sg):
        pltpu.make_async_copy(lines_hbm.at[pl.ds(0, CHUNK)], buf, sg).wait()

    def fire_o(j, buf, so):
        pltpu.async_copy(buf, out_hbm.at[pl.ds(base + j * CHUNK, CHUNK)], so)

    def wait_o(buf, so):
        pltpu.make_async_copy(buf, out_hbm.at[pl.ds(0, CHUNK)], so).wait()

    fire_g(0, b0, sg0)
    fire_g(1, b1, sg1)

    def body(j, _):
        s0 = 2 * j
        wait_g(b0, sg0)

        @pl.when(j > 0)
        def _():
            wait_o(ob0, so0)

        select(s0, b0, ob0)
        fire_o(s0, ob0, so0)

        @pl.when(j < N_PAIR - 1)
        def _():
            fire_g(s0 + 2, b0, sg0)

        wait_g(b1, sg1)

        @pl.when(j > 0)
        def _():
            wait_o(ob1, so1)

        select(s0 + 1, b1, ob1)
        fire_o(s0 + 1, ob1, so1)

        @pl.when(j < N_PAIR - 1)
        def _():
            fire_g(s0 + 3, b1, sg1)
        return 0

    lax.fori_loop(0, N_PAIR, body, 0)
    wait_o(ob0, so0)
    wait_o(ob1, so1)


def kernel(indices, table):
    idx = indices.astype(jnp.int32).reshape(NW, PER_W)
    tail = table[MAIN:].reshape(TAIL // 2, 128)
    lines = _transpose_kernel(table.T, tail)
    out = _gather_kernel(idx, lines)
    return out[:, :DIM].reshape(BATCH, HIST, DIM)


# speedup vs baseline: 1.2725x; 1.2725x over previous
import functools

import jax
import jax.numpy as jnp
from jax import lax
from jax.experimental import pallas as pl
from jax.experimental.pallas import tpu as pltpu
from jax.experimental.pallas import tpu_sc as plsc

BATCH = 4096
HIST = 20
DIM = 64
TOTAL = BATCH * HIST
NUM_EMB = 1000000
NUM_CORES = 2
NUM_SUBCORES = 16
NW = NUM_CORES * NUM_SUBCORES
PER_W = TOTAL // NW
CHUNK = 128
LANES = 16

TCH = 128
STEPS = (NUM_EMB // TCH) // NW
MAIN = STEPS * NW * TCH
TAIL = NUM_EMB - MAIN
N_CHUNK = PER_W // CHUNK
PACKED = NUM_EMB // 2
N_PAIR = N_CHUNK // 2

_mesh = plsc.VectorSubcoreMesh(core_axis_name="c", subcore_axis_name="s")
_params = pltpu.CompilerParams(use_tc_tiling_on_sc=True,
                               needs_layout_passes=False)


@functools.partial(
    pl.kernel,
    mesh=_mesh,
    out_type=jax.ShapeDtypeStruct((PACKED, 128), jnp.float32),
    scratch_types=[
        pltpu.VMEM((DIM, TCH), jnp.float32),
        pltpu.VMEM((DIM, TCH), jnp.float32),
        pltpu.VMEM((TCH // 2, 128), jnp.float32),
        pltpu.VMEM((TCH // 2, 128), jnp.float32),
        pltpu.SemaphoreType.DMA,
        pltpu.SemaphoreType.DMA,
        pltpu.SemaphoreType.DMA,
        pltpu.SemaphoreType.DMA,
    ],
    compiler_params=_params,
)
def _transpose_kernel(tT_hbm, tail_hbm, scratch_hbm,
                      tin0, tin1, tout0, tout1, si0, si1, so0, so1):
    wid = lax.axis_index("s") * NUM_CORES + lax.axis_index("c")
    lo = wid * STEPS
    iota = lax.iota(jnp.int32, LANES)
    one = jnp.full((LANES,), 1, jnp.int32)
    dimv = jnp.full((LANES,), DIM, jnp.int32)

    def fire_in(step, tin, si):
        pltpu.async_copy(tT_hbm.at[:, pl.ds(step * TCH, TCH)], tin, si)

    def wait_in(tin, si):
        pltpu.make_async_copy(tT_hbm.at[:, pl.ds(0, TCH)], tin, si).wait()

    def fire_out(step, tout, so):
        pltpu.async_copy(
            tout, scratch_hbm.at[pl.ds(step * (TCH // 2), TCH // 2)], so)

    def wait_out(tout, so):
        pltpu.make_async_copy(tout, scratch_hbm.at[pl.ds(0, TCH // 2)],
                              so).wait()

    def shuffle(tin, tout):
        @plsc.parallel_loop(0, TCH, step=LANES, unroll=4)
        def _(i0):
            for f0 in range(0, DIM, LANES):
                rowv = f0 + iota
                for k in range(LANES):
                    colv = i0 + lax.bitwise_and(iota + k, LANES - 1)
                    vals = plsc.load_gather(tin, [rowv, colv])
                    prow = lax.shift_right_logical(colv, one)
                    pcol = lax.mul(lax.bitwise_and(colv, one), dimv) + rowv
                    plsc.store_scatter(tout, [prow, pcol], vals)

    fire_in(lo, tin0, si0)
    fire_in(lo + 1, tin1, si1)

    def body(g, _):
        s0 = lo + 2 * g
        wait_in(tin0, si0)

        @pl.when(g > 0)
        def _():
            wait_out(tout0, so0)

        shuffle(tin0, tout0)
        fire_out(s0, tout0, so0)
        fire_in(s0 + 2, tin0, si0)

        wait_in(tin1, si1)

        @pl.when(g > 0)
        def _():
            wait_out(tout1, so1)

        shuffle(tin1, tout1)
        fire_out(s0 + 1, tout1, so1)
        fire_in(s0 + 3, tin1, si1)
        return 0

    lax.fori_loop(0, STEPS // 2, body, 0)
    wait_in(tin0, si0)
    wait_in(tin1, si1)
    wait_out(tout0, so0)
    wait_out(tout1, so1)

    @pl.when(wid == NW - 1)
    def _():
        half = TCH // 2
        for t in range(4):
            pltpu.sync_copy(tail_hbm.at[pl.ds(t * half, half)], tout0)
            pltpu.sync_copy(
                tout0, scratch_hbm.at[pl.ds(MAIN // 2 + t * half, half)])
        rest = TAIL // 2 - 4 * half
        pltpu.sync_copy(tail_hbm.at[pl.ds(4 * half, rest)],
                        tout0.at[pl.ds(0, rest)])
        pltpu.sync_copy(tout0.at[pl.ds(0, rest)],
                        scratch_hbm.at[pl.ds(MAIN // 2 + 4 * half, rest)])


@functools.partial(
    pl.kernel,
    mesh=_mesh,
    out_type=jax.ShapeDtypeStruct((TOTAL, 128), jnp.float32),
    scratch_types=[
        pltpu.VMEM((PER_W,), jnp.int32),
        pltpu.VMEM((PER_W,), jnp.int32),
        pltpu.VMEM((CHUNK, 128), jnp.float32),
        pltpu.VMEM((CHUNK, 128), jnp.float32),
        pltpu.VMEM((CHUNK, 128), jnp.float32),
        pltpu.VMEM((CHUNK, 128), jnp.float32),
        pltpu.SemaphoreType.DMA,
        pltpu.SemaphoreType.DMA,
        pltpu.SemaphoreType.DMA,
        pltpu.SemaphoreType.DMA,
    ],
    compiler_params=_params,
)
def _gather_kernel(idx_hbm, lines_hbm, out_hbm, idx_v, pidx_v, b0, b1,
                   ob0, ob1, sg0, sg1, so0, so1):
    wid = lax.axis_index("s") * NUM_CORES + lax.axis_index("c")
    base = wid * PER_W
    iota = lax.iota(jnp.int32, LANES)
    one = jnp.full((LANES,), 1, jnp.int32)
    dimv = jnp.full((LANES,), DIM, jnp.int32)
    pltpu.sync_copy(idx_hbm.at[wid], idx_v)

    @plsc.parallel_loop(0, PER_W, step=LANES, unroll=4)
    def _(i):
        pidx_v[pl.ds(i, LANES)] = lax.shift_right_logical(
            idx_v[pl.ds(i, LANES)], one)

    def select(j, buf, ob):
        off = j * CHUNK

        @plsc.parallel_loop(0, CHUNK, step=LANES, unroll=4)
        def _(r0):
            rows = r0 + iota
            hvec = lax.mul(
                lax.bitwise_and(idx_v[pl.ds(off + r0, LANES)], one), dimv)
            for f0 in range(0, DIM, LANES):
                for k in range(LANES):
                    fv = f0 + lax.bitwise_and(iota + k, LANES - 1)
                    vals = plsc.load_gather(buf, [rows, hvec + fv])
                    plsc.store_scatter(ob, [rows, fv], vals)

    def fire_g(j, buf, sg):
        pltpu.async_copy(lines_hbm.at[pidx_v.at[pl.ds(j * CHUNK, CHUNK)]],
                         buf, sg)

    def wait_g(buf, sg):
        pltpu.make_async_copy(lines_hbm.at[pl.ds(0, CHUNK)], buf, sg).wait()

    def fire_o(j, buf, so):
        pltpu.async_copy(buf, out_hbm.at[pl.ds(base + j * CHUNK, CHUNK)], so)

    def wait_o(buf, so):
        pltpu.make_async_copy(buf, out_hbm.at[pl.ds(0, CHUNK)], so).wait()

    fire_g(0, b0, sg0)
    fire_g(1, b1, sg1)

    def body(j, _):
        s0 = 2 * j
        wait_g(b0, sg0)

        @pl.when(j > 0)
        def _():
            wait_o(ob0, so0)

        select(s0, b0, ob0)
        fire_o(s0, ob0, so0)

        @pl.when(j < N_PAIR - 1)
        def _():
            fire_g(s0 + 2, b0, sg0)

        wait_g(b1, sg1)

        @pl.when(j > 0)
        def _():
            wait_o(ob1, so1)

        select(s0 + 1, b1, ob1)
        fire_o(s0 + 1, ob1, so1)

        @pl.when(j < N_PAIR - 1)
        def _():
            fire_g(s0 + 3, b1, sg1)
        return 0

    lax.fori_loop(0, N_PAIR, body, 0)
    wait_o(ob0, so0)
    wait_o(ob1, so1)


def kernel(indices, table):
    idx = indices.astype(jnp.int32).reshape(NW, PER_W)
    tail = table[MAIN:].reshape(TAIL // 2, 128)
    lines = _transpose_kernel(table.T, tail)
    out = _gather_kernel(idx, lines)
    return out[:, :DIM].reshape(BATCH, HIST, DIM)
